# baseline (speedup 1.0000x reference)
import functools

import jax
import jax.numpy as jnp
from jax import lax
from jax.experimental import pallas as pl
from jax.experimental.pallas import tpu as pltpu
from jax.experimental.pallas import tpu_sc as plsc

NW = 32
RB = 128
NB = 4


@functools.lru_cache(maxsize=None)
def _build(n: int, v: int, d: int):
    assert n % 8 == 0 and v % 16 == 0
    q8, r8 = divmod(n // 8, NW)
    cb = q8 * 8
    ca = cb + 8
    fa, pa = divmod(ca, RB)
    fb, pb = divmod(cb, RB)
    assert fa >= NB and fb >= NB
    off_b = r8 * ca

    mesh = plsc.VectorSubcoreMesh(core_axis_name="c", subcore_axis_name="s")

    @functools.partial(
        pl.kernel,
        mesh=mesh,
        out_type=jax.ShapeDtypeStruct((n, d), jnp.float32),
        scratch_types=[
            pltpu.VMEM((ca,), jnp.int32),
            pltpu.VMEM((NB, RB, d), jnp.float32),
            pltpu.VMEM_SHARED((v, d), jnp.float32),
        ]
        + [pltpu.SemaphoreType.DMA] * (2 * NB + 1),
    )
    def gather_kernel(idx_hbm, table_hbm, out_hbm, idx_v, rows_v, table_sh,
                      *sems):
        gsem = sems[:NB]
        wsem = sems[NB : 2 * NB]
        tsem = sems[2 * NB]
        cid = lax.axis_index("c")
        sid = lax.axis_index("s")
        wid = sid * 2 + cid
        is_a = wid < r8
        nfull = jnp.where(is_a, fa, fb)
        base = jnp.where(is_a, wid * ca, off_b + (wid - r8) * cb)
        base = pl.multiple_of(base, 8)

        vt = v // 16
        pltpu.async_copy(
            table_hbm.at[pl.ds(sid * vt, vt)],
            table_sh.at[pl.ds(sid * vt, vt)],
            tsem,
        )

        def stage_idx(count):
            pltpu.async_copy(
                idx_hbm.at[pl.ds(base, count)],
                idx_v.at[pl.ds(0, count)],
                gsem[1],
            )
            pltpu.make_async_copy(
                idx_hbm.at[pl.ds(base, count)],
                idx_v.at[pl.ds(0, count)],
                gsem[1],
            ).wait()

        @pl.when(is_a)
        def _():
            stage_idx(ca)

        @pl.when(jnp.logical_not(is_a))
        def _():
            stage_idx(cb)

        def start_gather(s, b):
            pltpu.async_copy(
                table_sh.at[idx_v.at[pl.ds(s * RB, RB)]], rows_v.at[b], gsem[b]
            )

        def wait_gather(s, b):
            pltpu.make_async_copy(
                table_sh.at[idx_v.at[pl.ds(s * RB, RB)]], rows_v.at[b], gsem[b]
            ).wait()

        def start_write(s, b):
            pltpu.async_copy(
                rows_v.at[b], out_hbm.at[pl.ds(base + s * RB, RB)], wsem[b]
            )

        def wait_write(b):
            pltpu.make_async_copy(
                rows_v.at[b], out_hbm.at[pl.ds(0, RB)], wsem[b]
            ).wait()

        for b in range(NB):
            pltpu.async_copy(
                table_hbm.at[idx_v.at[pl.ds(b * RB, RB)]], rows_v.at[b], gsem[b]
            )

        pltpu.make_async_copy(
            table_hbm.at[pl.ds(sid * vt, vt)],
            table_sh.at[pl.ds(sid * vt, vt)],
            tsem,
        ).wait()
        plsc.subcore_barrier()

        def body(s, _):
            def step(c):
                cg = (c + 2) % NB
                @pl.when((s >= 2) & (s + 2 < nfull))
                def _():
                    wait_write(cg)
                    start_gather(s + 2, cg)

                wait_gather(s, c)
                start_write(s, c)

            cur = lax.rem(s, NB)
            for c in range(NB):
                @pl.when(cur == c)
                def _(c=c):
                    step(c)

            return _

        lax.fori_loop(0, nfull, body, None)

        def epilogue(f, p):
            c_p = f % NB
            if p:
                wait_write(c_p)
                pltpu.async_copy(
                    table_sh.at[idx_v.at[pl.ds(f * RB, p)]],
                    rows_v.at[c_p].at[pl.ds(0, p)],
                    gsem[c_p],
                )
                pltpu.make_async_copy(
                    table_sh.at[idx_v.at[pl.ds(f * RB, p)]],
                    rows_v.at[c_p].at[pl.ds(0, p)],
                    gsem[c_p],
                ).wait()
                pltpu.async_copy(
                    rows_v.at[c_p].at[pl.ds(0, p)],
                    out_hbm.at[pl.ds(base + f * RB, p)],
                    wsem[c_p],
                )
            for b in range(NB):
                if p and b == c_p:
                    continue
                wait_write(b)
            if p:
                pltpu.make_async_copy(
                    rows_v.at[c_p].at[pl.ds(0, p)],
                    out_hbm.at[pl.ds(0, p)],
                    wsem[c_p],
                ).wait()

        @pl.when(is_a)
        def _():
            epilogue(fa, pa)

        @pl.when(jnp.logical_not(is_a))
        def _():
            epilogue(fb, pb)

    return gather_kernel


def kernel(x, inf_embed):
    n = x.shape[0]
    v, d = inf_embed.shape
    idx = x.astype(jnp.int32)
    return _build(n, v, d)(idx, inf_embed)


# speedup vs baseline: 1.0790x; 1.0790x over previous
import functools

import jax
import jax.numpy as jnp
from jax import lax
from jax.experimental import pallas as pl
from jax.experimental.pallas import tpu as pltpu
from jax.experimental.pallas import tpu_sc as plsc

NW = 32
RB = 128
NB = 4


@functools.lru_cache(maxsize=None)
def _build(n: int, v: int, d: int):
    assert n % 8 == 0 and v % 16 == 0
    q8, r8 = divmod(n // 8, NW)
    cb = q8 * 8
    ca = cb + 8
    fa, pa = divmod(ca, RB)
    fb, pb = divmod(cb, RB)
    assert fa >= NB and fb >= NB
    off_b = r8 * ca

    mesh = plsc.VectorSubcoreMesh(core_axis_name="c", subcore_axis_name="s")

    @functools.partial(
        pl.kernel,
        mesh=mesh,
        out_type=jax.ShapeDtypeStruct((n, d), jnp.float32),
        scratch_types=[
            pltpu.VMEM((ca,), jnp.int32),
            pltpu.VMEM((NB, RB, d), jnp.float32),
            pltpu.VMEM_SHARED((v, d), jnp.float32),
        ]
        + [pltpu.SemaphoreType.DMA] * (2 * NB + 1),
    )
    def gather_kernel(idx_hbm, table_hbm, out_hbm, idx_v, rows_v, table_sh,
                      *sems):
        gsem = sems[:NB]
        wsem = sems[NB : 2 * NB]
        tsem = sems[2 * NB]
        cid = lax.axis_index("c")
        sid = lax.axis_index("s")
        wid = sid * 2 + cid
        is_a = wid < r8
        nfull = jnp.where(is_a, fa, fb)
        base = jnp.where(is_a, wid * ca, off_b + (wid - r8) * cb)
        base = pl.multiple_of(base, 8)

        vt = v // 16
        pltpu.async_copy(
            table_hbm.at[pl.ds(sid * vt, vt)],
            table_sh.at[pl.ds(sid * vt, vt)],
            tsem,
        )

        def stage_idx(count):
            pltpu.async_copy(
                idx_hbm.at[pl.ds(base, count)],
                idx_v.at[pl.ds(0, count)],
                gsem[1],
            )
            pltpu.make_async_copy(
                idx_hbm.at[pl.ds(base, count)],
                idx_v.at[pl.ds(0, count)],
                gsem[1],
            ).wait()

        @pl.when(is_a)
        def _():
            stage_idx(ca)

        @pl.when(jnp.logical_not(is_a))
        def _():
            stage_idx(cb)

        def start_gather(s, b):
            pltpu.async_copy(
                table_sh.at[idx_v.at[pl.ds(s * RB, RB)]], rows_v.at[b], gsem[b]
            )

        def wait_gather(s, b):
            pltpu.make_async_copy(
                table_sh.at[idx_v.at[pl.ds(s * RB, RB)]], rows_v.at[b], gsem[b]
            ).wait()

        def start_write(s, b):
            pltpu.async_copy(
                rows_v.at[b], out_hbm.at[pl.ds(base + s * RB, RB)], wsem[b]
            )

        def wait_write(b):
            pltpu.make_async_copy(
                rows_v.at[b], out_hbm.at[pl.ds(0, RB)], wsem[b]
            ).wait()

        pltpu.make_async_copy(
            table_hbm.at[pl.ds(sid * vt, vt)],
            table_sh.at[pl.ds(sid * vt, vt)],
            tsem,
        ).wait()
        plsc.subcore_barrier()

        for b in range(NB):
            start_gather(b, b)

        def body(s, _):
            def step(c):
                cg = (c + 2) % NB
                @pl.when((s >= 2) & (s + 2 < nfull))
                def _():
                    wait_write(cg)
                    start_gather(s + 2, cg)

                wait_gather(s, c)
                start_write(s, c)

            cur = lax.rem(s, NB)
            for c in range(NB):
                @pl.when(cur == c)
                def _(c=c):
                    step(c)

            return _

        lax.fori_loop(0, nfull, body, None)

        def epilogue(f, p):
            c_p = f % NB
            if p:
                wait_write(c_p)
                pltpu.async_copy(
                    table_sh.at[idx_v.at[pl.ds(f * RB, p)]],
                    rows_v.at[c_p].at[pl.ds(0, p)],
                    gsem[c_p],
                )
                pltpu.make_async_copy(
                    table_sh.at[idx_v.at[pl.ds(f * RB, p)]],
                    rows_v.at[c_p].at[pl.ds(0, p)],
                    gsem[c_p],
                ).wait()
                pltpu.async_copy(
                    rows_v.at[c_p].at[pl.ds(0, p)],
                    out_hbm.at[pl.ds(base + f * RB, p)],
                    wsem[c_p],
                )
            for b in range(NB):
                if p and b == c_p:
                    continue
                wait_write(b)
            if p:
                pltpu.make_async_copy(
                    rows_v.at[c_p].at[pl.ds(0, p)],
                    out_hbm.at[pl.ds(0, p)],
                    wsem[c_p],
                ).wait()

        @pl.when(is_a)
        def _():
            epilogue(fa, pa)

        @pl.when(jnp.logical_not(is_a))
        def _():
            epilogue(fb, pb)

    return gather_kernel


def kernel(x, inf_embed):
    n = x.shape[0]
    v, d = inf_embed.shape
    idx = x.astype(jnp.int32)
    return _build(n, v, d)(idx, inf_embed)


# speedup vs baseline: 1.0808x; 1.0017x over previous
import functools

import jax
import jax.numpy as jnp
from jax import lax
from jax.experimental import pallas as pl
from jax.experimental.pallas import tpu as pltpu
from jax.experimental.pallas import tpu_sc as plsc

NW = 32
RB = 128
NB = 6


@functools.lru_cache(maxsize=None)
def _build(n: int, v: int, d: int):
    assert n % 8 == 0 and v % 16 == 0
    q8, r8 = divmod(n // 8, NW)
    cb = q8 * 8
    ca = cb + 8
    fa, pa = divmod(ca, RB)
    fb, pb = divmod(cb, RB)
    assert fa >= NB and fb >= NB
    off_b = r8 * ca

    mesh = plsc.VectorSubcoreMesh(core_axis_name="c", subcore_axis_name="s")

    @functools.partial(
        pl.kernel,
        mesh=mesh,
        out_type=jax.ShapeDtypeStruct((n, d), jnp.float32),
        scratch_types=[
            pltpu.VMEM((ca,), jnp.int32),
            pltpu.VMEM((NB, RB, d), jnp.float32),
            pltpu.VMEM_SHARED((v, d), jnp.float32),
        ]
        + [pltpu.SemaphoreType.DMA] * (2 * NB + 1),
    )
    def gather_kernel(idx_hbm, table_hbm, out_hbm, idx_v, rows_v, table_sh,
                      *sems):
        gsem = sems[:NB]
        wsem = sems[NB : 2 * NB]
        tsem = sems[2 * NB]
        cid = lax.axis_index("c")
        sid = lax.axis_index("s")
        wid = sid * 2 + cid
        is_a = wid < r8
        nfull = jnp.where(is_a, fa, fb)
        base = jnp.where(is_a, wid * ca, off_b + (wid - r8) * cb)
        base = pl.multiple_of(base, 8)

        vt = v // 16
        pltpu.async_copy(
            table_hbm.at[pl.ds(sid * vt, vt)],
            table_sh.at[pl.ds(sid * vt, vt)],
            tsem,
        )

        def stage_idx(count):
            pltpu.async_copy(
                idx_hbm.at[pl.ds(base, count)],
                idx_v.at[pl.ds(0, count)],
                gsem[1],
            )
            pltpu.make_async_copy(
                idx_hbm.at[pl.ds(base, count)],
                idx_v.at[pl.ds(0, count)],
                gsem[1],
            ).wait()

        @pl.when(is_a)
        def _():
            stage_idx(ca)

        @pl.when(jnp.logical_not(is_a))
        def _():
            stage_idx(cb)

        def start_gather(s, b):
            pltpu.async_copy(
                table_sh.at[idx_v.at[pl.ds(s * RB, RB)]], rows_v.at[b], gsem[b]
            )

        def wait_gather(s, b):
            pltpu.make_async_copy(
                table_sh.at[idx_v.at[pl.ds(s * RB, RB)]], rows_v.at[b], gsem[b]
            ).wait()

        def start_write(s, b):
            pltpu.async_copy(
                rows_v.at[b], out_hbm.at[pl.ds(base + s * RB, RB)], wsem[b]
            )

        def wait_write(b):
            pltpu.make_async_copy(
                rows_v.at[b], out_hbm.at[pl.ds(0, RB)], wsem[b]
            ).wait()

        pltpu.make_async_copy(
            table_hbm.at[pl.ds(sid * vt, vt)],
            table_sh.at[pl.ds(sid * vt, vt)],
            tsem,
        ).wait()
        plsc.subcore_barrier()

        for b in range(NB):
            start_gather(b, b)

        def body(s, _):
            def step(c):
                cg = (c + 2) % NB
                @pl.when((s >= NB - 2) & (s + 2 < nfull))
                def _():
                    wait_write(cg)
                    start_gather(s + 2, cg)

                wait_gather(s, c)
                start_write(s, c)

            cur = lax.rem(s, NB)
            for c in range(NB):
                @pl.when(cur == c)
                def _(c=c):
                    step(c)

            return _

        lax.fori_loop(0, nfull, body, None)

        def epilogue(f, p):
            c_p = f % NB
            if p:
                wait_write(c_p)
                pltpu.async_copy(
                    table_sh.at[idx_v.at[pl.ds(f * RB, p)]],
                    rows_v.at[c_p].at[pl.ds(0, p)],
                    gsem[c_p],
                )
                pltpu.make_async_copy(
                    table_sh.at[idx_v.at[pl.ds(f * RB, p)]],
                    rows_v.at[c_p].at[pl.ds(0, p)],
                    gsem[c_p],
                ).wait()
                pltpu.async_copy(
                    rows_v.at[c_p].at[pl.ds(0, p)],
                    out_hbm.at[pl.ds(base + f * RB, p)],
                    wsem[c_p],
                )
            for b in range(NB):
                if p and b == c_p:
                    continue
                wait_write(b)
            if p:
                pltpu.make_async_copy(
                    rows_v.at[c_p].at[pl.ds(0, p)],
                    out_hbm.at[pl.ds(0, p)],
                    wsem[c_p],
                ).wait()

        @pl.when(is_a)
        def _():
            epilogue(fa, pa)

        @pl.when(jnp.logical_not(is_a))
        def _():
            epilogue(fb, pb)

    return gather_kernel


def kernel(x, inf_embed):
    n = x.shape[0]
    v, d = inf_embed.shape
    idx = x.astype(jnp.int32)
    return _build(n, v, d)(idx, inf_embed)
